# baseline (speedup 1.0000x reference)
import functools

import jax
import jax.numpy as jnp
from jax import lax
from jax.experimental import pallas as pl
from jax.experimental.pallas import tpu as pltpu
from jax.experimental.pallas import tpu_sc as plsc

N = 10000
NP = 10240
E = 320000
G = 128
H = 4
D1 = 256
D2 = 64
F1 = H * D1
CHUNK = 128
NCHUNK = F1 // CHUNK

NC = 2
NS = 16
NW = NC * NS
EPT = E // NW
ROWS_T = NP // NS
ZR = 128

RB = 1024
NBLK = NP // RB

EB1 = 2000
EB2 = 80

_f32 = jnp.float32
_i32 = jnp.int32


def _k1_body(x_ref, w1_ref, a1_ref, *out_refs):
    xb = x_ref[...]
    h = jnp.dot(xb, w1_ref[...], preferred_element_type=_f32)
    for c in range(NCHUNK):
        out_refs[c][...] = h[:, c * CHUNK:(c + 1) * CHUNK]
    out_refs[NCHUNK][...] = lax.dot_general(
        a1_ref[...], h, (((0,), (1,)), ((), ())),
        preferred_element_type=_f32, precision=lax.Precision.HIGHEST)


def _k1_call(x, W1, A1):
    return pl.pallas_call(
        _k1_body,
        grid=(NBLK,),
        in_specs=[
            pl.BlockSpec((RB, G), lambda i: (i, 0)),
            pl.BlockSpec((G, F1), lambda i: (0, 0)),
            pl.BlockSpec((F1, 16), lambda i: (0, 0)),
        ],
        out_specs=[pl.BlockSpec((RB, CHUNK), lambda i: (i, 0))] * NCHUNK
        + [pl.BlockSpec((16, RB), lambda i: (0, i))],
        out_shape=[jax.ShapeDtypeStruct((NP, CHUNK), _f32)] * NCHUNK
        + [jax.ShapeDtypeStruct((16, NP), _f32)],
    )(x, W1, A1)


def _att_proj_body(x_ref, va_ref, out_ref):
    out_ref[...] = lax.dot_general(
        va_ref[...], x_ref[...], (((0,), (1,)), ((), ())),
        preferred_element_type=_f32, precision=lax.Precision.HIGHEST)


def _att_proj_call(x, Va):
    k = x.shape[1]
    return pl.pallas_call(
        _att_proj_body,
        grid=(1,),
        in_specs=[
            pl.BlockSpec((NP, k), lambda i: (0, 0)),
            pl.BlockSpec((k, 16), lambda i: (0, 0)),
        ],
        out_specs=pl.BlockSpec((16, NP), lambda i: (0, 0)),
        out_shape=jax.ShapeDtypeStruct((16, NP), _f32),
    )(x, Va)


def _k3_body(*refs):
    s_refs = refs[:NCHUNK]
    dp_ref, w2_ref, b1_ref, h2pre_ref = refs[NCHUNK:]
    dpv = dp_ref[0] + dp_ref[1]
    parts = []
    for c in range(NCHUNK):
        sc = s_refs[c][0] + s_refs[c][1]
        h = c // (NCHUNK // H)
        deninv = 1.0 / (dpv[:, h * 32] + 1e-16)
        parts.append(sc * deninv[:, None])
    h1 = jnp.concatenate(parts, axis=1) + b1_ref[...]
    h1 = jnp.where(h1 > 0, h1, jnp.exp(jnp.minimum(h1, 0.0)) - 1.0)
    h2pre_ref[...] = jnp.dot(h1, w2_ref[...], preferred_element_type=_f32)


def _k3_call(s_chunks, dp, W2, b1r):
    return pl.pallas_call(
        _k3_body,
        grid=(NBLK,),
        in_specs=[pl.BlockSpec((NC, RB, CHUNK), lambda i: (0, i, 0))] * NCHUNK
        + [
            pl.BlockSpec((NC, RB, CHUNK), lambda i: (0, i, 0)),
            pl.BlockSpec((F1, D2), lambda i: (0, 0)),
            pl.BlockSpec((1, F1), lambda i: (0, 0)),
        ],
        out_specs=pl.BlockSpec((RB, D2), lambda i: (i, 0)),
        out_shape=jax.ShapeDtypeStruct((NP, D2), _f32),
    )(*s_chunks, dp, W2, b1r)


def _k5a_body(s2_ref, dp2_ref, wd_ref, bd_ref, b2_ref,
              rep_ref, hd_ref, s1_ref, sq_ref):
    den = dp2_ref[...].sum(axis=(0, 2)) * (1.0 / 16.0)
    ssum = s2_ref[0] + s2_ref[1]
    rep = ssum * (1.0 / (den + 1e-16))[:, None] + b2_ref[...]
    rep_ref[...] = rep
    hd = jnp.dot(rep, wd_ref[...], preferred_element_type=_f32) + bd_ref[...]
    hd_ref[...] = hd
    i = pl.program_id(0)

    @pl.when(i == 0)
    def _():
        s1_ref[...] = jnp.zeros_like(s1_ref)
        sq_ref[...] = jnp.zeros_like(sq_ref)

    row = lax.broadcasted_iota(_i32, (RB, 1), 0) + i * RB
    hdm = jnp.where(row < N, hd, 0.0)
    s1_ref[...] += hdm.sum(axis=0, keepdims=True)
    sq_ref[...] += (hdm * hdm).sum(axis=0, keepdims=True)


def _k5a_call(S2, dp2, Wd, bdr, b2r):
    return pl.pallas_call(
        _k5a_body,
        grid=(NBLK,),
        in_specs=[
            pl.BlockSpec((NC, RB, D2), lambda i: (0, i, 0)),
            pl.BlockSpec((NC, RB, 16), lambda i: (0, i, 0)),
            pl.BlockSpec((D2, D1), lambda i: (0, 0)),
            pl.BlockSpec((1, D1), lambda i: (0, 0)),
            pl.BlockSpec((1, D2), lambda i: (0, 0)),
        ],
        out_specs=[
            pl.BlockSpec((RB, D2), lambda i: (i, 0)),
            pl.BlockSpec((RB, D1), lambda i: (i, 0)),
            pl.BlockSpec((1, D1), lambda i: (0, 0)),
            pl.BlockSpec((1, D1), lambda i: (0, 0)),
        ],
        out_shape=[
            jax.ShapeDtypeStruct((NP, D2), _f32),
            jax.ShapeDtypeStruct((NP, D1), _f32),
            jax.ShapeDtypeStruct((1, D1), _f32),
            jax.ShapeDtypeStruct((1, D1), _f32),
        ],
    )(S2, dp2, Wd, bdr, b2r)


def _k5c_body(hd_ref, s1_ref, sq_ref, g_ref, be_ref,
              wm_ref, bm_ref, wdi_ref, bdi_ref, wp_ref, bp_ref,
              mean_ref, disp_ref, pi_ref):
    mu = s1_ref[...] * (1.0 / N)
    var = sq_ref[...] * (1.0 / N) - mu * mu
    scale = lax.rsqrt(var + 1e-5) * g_ref[...]
    hn = (hd_ref[...] - mu) * scale + be_ref[...]
    hn = jnp.maximum(hn, 0.0)
    m = jnp.dot(hn, wm_ref[...], preferred_element_type=_f32) + bm_ref[...]
    mean_ref[...] = jnp.clip(jnp.exp(m), 1e-5, 1e6)
    d = jnp.dot(hn, wdi_ref[...], preferred_element_type=_f32) + bdi_ref[...]
    sp = jnp.maximum(d, 0.0) + jnp.log(1.0 + jnp.exp(-jnp.abs(d)))
    disp_ref[...] = jnp.clip(sp, 1e-4, 1e4)
    p = jnp.dot(hn, wp_ref[...], preferred_element_type=_f32) + bp_ref[...]
    pi_ref[...] = 1.0 / (1.0 + jnp.exp(-p))


def _k5c_call(hd, s1, sq, gr, ber, Wm, bmr, Wdi, bdir, Wp, bpr):
    return pl.pallas_call(
        _k5c_body,
        grid=(NBLK,),
        in_specs=[
            pl.BlockSpec((RB, D1), lambda i: (i, 0)),
            pl.BlockSpec((1, D1), lambda i: (0, 0)),
            pl.BlockSpec((1, D1), lambda i: (0, 0)),
            pl.BlockSpec((1, D1), lambda i: (0, 0)),
            pl.BlockSpec((1, D1), lambda i: (0, 0)),
            pl.BlockSpec((D1, G), lambda i: (0, 0)),
            pl.BlockSpec((1, G), lambda i: (0, 0)),
            pl.BlockSpec((D1, G), lambda i: (0, 0)),
            pl.BlockSpec((1, G), lambda i: (0, 0)),
            pl.BlockSpec((D1, G), lambda i: (0, 0)),
            pl.BlockSpec((1, G), lambda i: (0, 0)),
        ],
        out_specs=[pl.BlockSpec((RB, G), lambda i: (i, 0))] * 3,
        out_shape=[jax.ShapeDtypeStruct((NP, G), _f32)] * 3,
    )(hd, s1, sq, gr, ber, Wm, bmr, Wdi, bdir, Wp, bpr)


_MESH = plsc.VectorSubcoreMesh(core_axis_name="c", subcore_axis_name="s")
_Z16 = None


def _wid():
    return lax.axis_index("s") * NC + lax.axis_index("c")


def _edge_w_kernel(a1t_hbm, src_hbm, dst_hbm, *rest):
    w_hbms = rest[:H]
    atabs = rest[H:H + 2 * H]
    srcv, dstv = rest[H + 2 * H:H + 2 * H + 2]
    wvs = rest[H + 2 * H + 2:]
    base = _wid() * EPT
    for t in range(2 * H):
        pltpu.sync_copy(a1t_hbm.at[t], atabs[t])

    def batch(j, carry):
        b0 = base + j * EB1
        pltpu.sync_copy(src_hbm.at[pl.ds(b0, EB1)], srcv)
        pltpu.sync_copy(dst_hbm.at[pl.ds(b0, EB1)], dstv)

        def grp(k, c2):
            s16 = srcv[pl.ds(k * 16, 16)]
            d16 = dstv[pl.ds(k * 16, 16)]
            for h in range(H):
                av = plsc.load_gather(atabs[h], [s16])
                bv = plsc.load_gather(atabs[h + H], [d16])
                t = av + bv
                w = jnp.exp(jnp.maximum(t, 0.2 * t))
                wvs[h][pl.ds(k * 16, 16)] = w
            return c2

        lax.fori_loop(0, EB1 // 16, grp, 0)
        for h in range(H):
            pltpu.sync_copy(wvs[h], w_hbms[h].at[pl.ds(b0, EB1)])
        return carry

    lax.fori_loop(0, EPT // EB1, batch, 0)


def _edge_w_call(a1t, src, dst):
    return pl.kernel(
        _edge_w_kernel,
        out_type=[jax.ShapeDtypeStruct((E,), _f32)] * H,
        mesh=_MESH,
        compiler_params=pltpu.CompilerParams(needs_layout_passes=False, use_tc_tiling_on_sc=False),
        scratch_types=[pltpu.VMEM((NP,), _f32)] * (2 * H) + [
            pltpu.VMEM((EB1,), _i32),
            pltpu.VMEM((EB1,), _i32),
        ] + [pltpu.VMEM((EB1,), _f32)] * H,
    )(a1t, src, dst)


def _zero_rows(ref, nrows, width):
    z = jnp.zeros((16,), _f32)

    def body(i, c):
        for k in range(width // 16):
            ref[i, pl.ds(k * 16, 16)] = z
        return c

    lax.fori_loop(0, nrows, body, 0)


def _agg1_kernel(*refs):
    h_refs = refs[:NCHUNK]
    src_hbm, dst_hbm = refs[NCHUNK:NCHUNK + 2]
    w_hbms = refs[NCHUNK + 2:NCHUNK + 2 + H]
    s_refs = refs[NCHUNK + 2 + H:2 * NCHUNK + 2 + H]
    dp_hbm = refs[2 * NCHUNK + 2 + H]
    rest = refs[2 * NCHUNK + 3 + H:]
    srcv_all, dstv_all = rest[0], rest[1]
    rows = rest[2:4]
    dsts = rest[4:6]
    wsm = rest[6:6 + H]
    sems = rest[6 + H:8 + H]
    ssems = rest[8 + H:10 + H]

    cid = lax.axis_index("c")
    sid = lax.axis_index("s")
    ebase = (sid * NC + cid) * EPT
    span0 = sid * ROWS_T
    NB = EPT // EB2

    pltpu.sync_copy(src_hbm.at[pl.ds(ebase, EPT)], srcv_all)
    pltpu.sync_copy(dst_hbm.at[pl.ds(ebase, EPT)], dstv_all)

    def _zero_fill():
        _zero_rows(rows[1], EB2, CHUNK)
        for r in range(ROWS_T // EB2):
            pltpu.sync_copy(rows[1], acc.at[pl.ds(span0 + r * EB2, EB2)])

    acc = rest[10 + H]
    _zero_fill()
    plsc.subcore_barrier()

    def _stage_dst(buf, j):
        for k in range(EB2 // 16):
            dsts[buf][pl.ds(k * 16, 16)] = \
                dstv_all[pl.ds(j * EB2 + k * 16, 16)]

    for c in range(NCHUNK + 1):
        if c < NCHUNK:
            head = c // (NCHUNK // H)
            h_ref = h_refs[c]
            s_ref = s_refs[c]

            def issue(j, buf):
                pltpu.async_copy(
                    w_hbms[head].at[pl.ds(ebase + j * EB2, EB2)],
                    wsm[buf], sems[buf])
                pltpu.async_copy(
                    h_ref.at[srcv_all.at[pl.ds(j * EB2, EB2)]],
                    rows[buf], sems[buf])

            def drain_scatter(buf):
                pltpu.make_async_copy(rows[buf], acc.at[dsts[buf]],
                                      ssems[buf]).wait()

            def process(j, buf):
                pltpu.make_async_copy(
                    w_hbms[head].at[pl.ds(ebase + j * EB2, EB2)],
                    wsm[buf], sems[buf]).wait()
                pltpu.make_async_copy(
                    h_ref.at[srcv_all.at[pl.ds(j * EB2, EB2)]],
                    rows[buf], sems[buf]).wait()
                _stage_dst(buf, j)

                def mul_edge(bb, c2):
                    wb = plsc.load_gather(
                        wsm[buf], [jnp.full((16,), bb, _i32)])
                    for f in range(CHUNK // 16):
                        rows[buf][bb, pl.ds(f * 16, 16)] = \
                            rows[buf][bb, pl.ds(f * 16, 16)] * wb
                    return c2

                lax.fori_loop(0, EB2, mul_edge, 0)
                pltpu.async_copy(rows[buf], acc.at[dsts[buf]], ssems[buf],
                                 add=True)

            issue(0, 0)
            issue(1, 1)

            def pair(k, c2):
                process(2 * k, 0)
                process(2 * k + 1, 1)
                drain_scatter(0)
                issue(2 * k + 2, 0)
                drain_scatter(1)

                @pl.when(2 * k + 3 < NB)
                def _():
                    issue(2 * k + 3, 1)

                return c2

            lax.fori_loop(0, (NB - 1) // 2, pair, 0)
            process(NB - 1, 0)
            drain_scatter(0)
        else:
            def dbatch(j, c2):
                for h in range(H):
                    pltpu.async_copy(
                        w_hbms[h].at[pl.ds(ebase + j * EB2, EB2)],
                        wsm[h], sems[0])
                for h in range(H):
                    pltpu.make_async_copy(
                        w_hbms[h].at[pl.ds(ebase + j * EB2, EB2)],
                        wsm[h], sems[0]).wait()
                _stage_dst(0, j)

                def wedge(bb, c3):
                    for h in range(H):
                        wb = plsc.load_gather(
                            wsm[h], [jnp.full((16,), bb, _i32)])
                        rows[0][bb, pl.ds(h * 32, 16)] = wb
                        rows[0][bb, pl.ds(h * 32 + 16, 16)] = wb
                    return c3

                lax.fori_loop(0, EB2, wedge, 0)
                pltpu.sync_copy(rows[0], acc.at[dsts[0]], add=True)
                return c2

            lax.fori_loop(0, NB, dbatch, 0)
            s_ref = dp_hbm

        plsc.subcore_barrier()
        pltpu.sync_copy(acc.at[pl.ds(span0, ROWS_T)],
                        s_ref.at[cid, pl.ds(span0, ROWS_T)])
        _zero_fill()
        plsc.subcore_barrier()


def _agg1_call(h_chunks, src, dst, w4):
    return pl.kernel(
        _agg1_kernel,
        out_type=[jax.ShapeDtypeStruct((NC, NP, CHUNK), _f32)] * NCHUNK
        + [jax.ShapeDtypeStruct((NC, NP, CHUNK), _f32)],
        mesh=_MESH,
        compiler_params=pltpu.CompilerParams(needs_layout_passes=False,
                                             use_tc_tiling_on_sc=False),
        scratch_types=[
            pltpu.VMEM((EPT,), _i32),
            pltpu.VMEM((EPT,), _i32),
            pltpu.VMEM((EB2, CHUNK), _f32),
            pltpu.VMEM((EB2, CHUNK), _f32),
            pltpu.VMEM((EB2,), _i32),
            pltpu.VMEM((EB2,), _i32),
        ] + [pltpu.VMEM((EB2,), _f32)] * H + [
            pltpu.SemaphoreType.DMA,
            pltpu.SemaphoreType.DMA,
            pltpu.SemaphoreType.DMA,
            pltpu.SemaphoreType.DMA,
            pltpu.VMEM_SHARED((NP, CHUNK), _f32),
        ],
    )(*h_chunks, src, dst, *w4)


def _agg2_kernel(h2_hbm, a2s_hbm, a2d_hbm, src_hbm, dst_hbm,
                 s2_hbm, dp2_hbm,
                 a2sv, a2dv, rows_v, srcv, dstv, wv, wrows, zb, zbd,
                 acc, dden, sem):
    cid = lax.axis_index("c")
    sid = lax.axis_index("s")
    ebase = (sid * NC + cid) * EPT
    span0 = sid * ROWS_T

    pltpu.sync_copy(a2s_hbm, a2sv)
    pltpu.sync_copy(a2d_hbm, a2dv)
    _zero_rows(zb, ZR, D2)
    _zero_rows(zbd, ZR, 16)
    for r in range(5):
        pltpu.sync_copy(zb, acc.at[pl.ds(span0 + r * ZR, ZR)])
        pltpu.sync_copy(zbd, dden.at[pl.ds(span0 + r * ZR, ZR)])
    plsc.subcore_barrier()

    def batch(j, carry):
        b0 = ebase + j * EB2
        pltpu.sync_copy(src_hbm.at[pl.ds(b0, EB2)], srcv)
        pltpu.sync_copy(dst_hbm.at[pl.ds(b0, EB2)], dstv)

        def grp(k, c2):
            s16 = srcv[pl.ds(k * 16, 16)]
            d16 = dstv[pl.ds(k * 16, 16)]
            av = plsc.load_gather(a2sv, [s16])
            bv = plsc.load_gather(a2dv, [d16])
            t = av + bv
            wv[pl.ds(k * 16, 16)] = jnp.exp(jnp.maximum(t, 0.2 * t))
            return c2

        lax.fori_loop(0, EB2 // 16, grp, 0)
        pltpu.async_copy(h2_hbm.at[srcv], rows_v, sem).wait()

        def mul_edge(b, c2):
            wb = plsc.load_gather(wv, [jnp.full((16,), b, _i32)])
            for f in range(D2 // 16):
                rows_v[b, pl.ds(f * 16, 16)] = \
                    rows_v[b, pl.ds(f * 16, 16)] * wb
            wrows[b] = wb
            return c2

        lax.fori_loop(0, EB2, mul_edge, 0)
        pltpu.sync_copy(rows_v, acc.at[dstv], add=True)
        pltpu.sync_copy(wrows, dden.at[dstv], add=True)
        return carry

    lax.fori_loop(0, EPT // EB2, batch, 0)
    plsc.subcore_barrier()
    pltpu.sync_copy(acc.at[pl.ds(span0, ROWS_T)],
                    s2_hbm.at[cid, pl.ds(span0, ROWS_T)])
    pltpu.sync_copy(dden.at[pl.ds(span0, ROWS_T)],
                    dp2_hbm.at[cid, pl.ds(span0, ROWS_T)])


def _agg2_call(h2pre, a2s, a2d, src, dst):
    return pl.kernel(
        _agg2_kernel,
        out_type=[
            jax.ShapeDtypeStruct((NC, NP, D2), _f32),
            jax.ShapeDtypeStruct((NC, NP, 16), _f32),
        ],
        mesh=_MESH,
        compiler_params=pltpu.CompilerParams(needs_layout_passes=False, use_tc_tiling_on_sc=False),
        scratch_types=[
            pltpu.VMEM((NP,), _f32),
            pltpu.VMEM((NP,), _f32),
            pltpu.VMEM((EB2, D2), _f32),
            pltpu.VMEM((EB2,), _i32),
            pltpu.VMEM((EB2,), _i32),
            pltpu.VMEM((EB2,), _f32),
            pltpu.VMEM((EB2, 16), _f32),
            pltpu.VMEM((ZR, D2), _f32),
            pltpu.VMEM((ZR, 16), _f32),
            pltpu.VMEM_SHARED((NP, D2), _f32),
            pltpu.VMEM_SHARED((NP, 16), _f32),
            pltpu.SemaphoreType.DMA,
        ],
    )(h2pre, a2s, a2d, src, dst)


def kernel(x, edge_index, W1, att_src1, att_dst1, b1, W2, att_src2, att_dst2,
           b2, Wd, bd, gamma, beta, Wm, bm, Wdi, bdi, Wp, bp):
    src = edge_index[0]
    dst = edge_index[1]

    eyeH = jnp.eye(H, dtype=_f32)
    a_s = (eyeH[:, None, :] * att_src1[:, :, None]).reshape(F1, H)
    a_d = (eyeH[:, None, :] * att_dst1[:, :, None]).reshape(F1, H)
    A1 = jnp.concatenate([a_s, a_d, jnp.zeros((F1, 8), _f32)], axis=1)
    At2 = jnp.concatenate(
        [att_src2.T, att_dst2.T, jnp.zeros((D2, 14), _f32)], axis=1)

    x_pad = jnp.pad(x, ((0, NP - N), (0, 0)))
    *h_chunks, a1t = _k1_call(x_pad, W1, A1)
    w4 = _edge_w_call(a1t, src, dst)
    *s_chunks, dp = _agg1_call(h_chunks, src, dst, w4)
    h2pre = _k3_call(s_chunks, dp, W2, b1.reshape(1, F1))
    a2t = _att_proj_call(h2pre, At2)
    S2, dp2 = _agg2_call(h2pre, a2t[0], a2t[1], src, dst)
    rep, hd, s1, sq = _k5a_call(S2, dp2, Wd, bd.reshape(1, D1),
                                b2.reshape(1, D2))
    mean, disp, pi = _k5c_call(
        hd, s1, sq, gamma.reshape(1, D1), beta.reshape(1, D1),
        Wm, bm.reshape(1, G), Wdi, bdi.reshape(1, G), Wp, bp.reshape(1, G))
    return (mean[:N], disp[:N], pi[:N], rep[:N])


# speedup vs baseline: 1.1495x; 1.1495x over previous
import functools

import jax
import jax.numpy as jnp
from jax import lax
from jax.experimental import pallas as pl
from jax.experimental.pallas import tpu as pltpu
from jax.experimental.pallas import tpu_sc as plsc

N = 10000
NP = 10240
E = 320000
G = 128
H = 4
D1 = 256
D2 = 64
F1 = H * D1
CHUNK = 128
NCHUNK = F1 // CHUNK

NC = 2
NS = 16
NW = NC * NS
EPT = E // NW
ROWS_T = NP // NS
ZR = 128

RB = 1024
NBLK = NP // RB

EB1 = 2000
EB2 = 80

_f32 = jnp.float32
_i32 = jnp.int32


def _k1_body(x_ref, w1_ref, a1_ref, *out_refs):
    xb = x_ref[...]
    h = jnp.dot(xb, w1_ref[...], preferred_element_type=_f32)
    for c in range(NCHUNK):
        out_refs[c][...] = h[:, c * CHUNK:(c + 1) * CHUNK]
    out_refs[NCHUNK][...] = lax.dot_general(
        a1_ref[...], h, (((0,), (1,)), ((), ())),
        preferred_element_type=_f32, precision=lax.Precision.HIGHEST)


def _k1_call(x, W1, A1):
    return pl.pallas_call(
        _k1_body,
        grid=(NBLK,),
        in_specs=[
            pl.BlockSpec((RB, G), lambda i: (i, 0)),
            pl.BlockSpec((G, F1), lambda i: (0, 0)),
            pl.BlockSpec((F1, 16), lambda i: (0, 0)),
        ],
        out_specs=[pl.BlockSpec((RB, CHUNK), lambda i: (i, 0))] * NCHUNK
        + [pl.BlockSpec((16, RB), lambda i: (0, i))],
        out_shape=[jax.ShapeDtypeStruct((NP, CHUNK), _f32)] * NCHUNK
        + [jax.ShapeDtypeStruct((16, NP), _f32)],
    )(x, W1, A1)


def _att_proj_body(x_ref, va_ref, out_ref):
    out_ref[...] = lax.dot_general(
        va_ref[...], x_ref[...], (((0,), (1,)), ((), ())),
        preferred_element_type=_f32, precision=lax.Precision.HIGHEST)


def _att_proj_call(x, Va):
    k = x.shape[1]
    return pl.pallas_call(
        _att_proj_body,
        grid=(1,),
        in_specs=[
            pl.BlockSpec((NP, k), lambda i: (0, 0)),
            pl.BlockSpec((k, 16), lambda i: (0, 0)),
        ],
        out_specs=pl.BlockSpec((16, NP), lambda i: (0, 0)),
        out_shape=jax.ShapeDtypeStruct((16, NP), _f32),
    )(x, Va)


def _k3_body(*refs):
    s_refs = refs[:NCHUNK]
    dp_ref, w2_ref, b1_ref, h2pre_ref = refs[NCHUNK:]
    dpv = dp_ref[0] + dp_ref[1]
    parts = []
    for c in range(NCHUNK):
        sc = s_refs[c][0] + s_refs[c][1]
        h = c // (NCHUNK // H)
        deninv = 1.0 / (dpv[:, h * 32] + 1e-16)
        parts.append(sc * deninv[:, None])
    h1 = jnp.concatenate(parts, axis=1) + b1_ref[...]
    h1 = jnp.where(h1 > 0, h1, jnp.exp(jnp.minimum(h1, 0.0)) - 1.0)
    h2pre_ref[...] = jnp.dot(h1, w2_ref[...], preferred_element_type=_f32)


def _k3_call(s_chunks, dp, W2, b1r):
    return pl.pallas_call(
        _k3_body,
        grid=(NBLK,),
        in_specs=[pl.BlockSpec((NC, RB, CHUNK), lambda i: (0, i, 0))] * NCHUNK
        + [
            pl.BlockSpec((NC, RB, CHUNK), lambda i: (0, i, 0)),
            pl.BlockSpec((F1, D2), lambda i: (0, 0)),
            pl.BlockSpec((1, F1), lambda i: (0, 0)),
        ],
        out_specs=pl.BlockSpec((RB, D2), lambda i: (i, 0)),
        out_shape=jax.ShapeDtypeStruct((NP, D2), _f32),
    )(*s_chunks, dp, W2, b1r)


def _k5a_body(s2_ref, dp2_ref, wd_ref, bd_ref, b2_ref,
              rep_ref, hd_ref, s1_ref, sq_ref):
    den = dp2_ref[...].sum(axis=(0, 2)) * (1.0 / 16.0)
    ssum = s2_ref[0] + s2_ref[1]
    rep = ssum * (1.0 / (den + 1e-16))[:, None] + b2_ref[...]
    rep_ref[...] = rep
    hd = jnp.dot(rep, wd_ref[...], preferred_element_type=_f32) + bd_ref[...]
    hd_ref[...] = hd
    i = pl.program_id(0)

    @pl.when(i == 0)
    def _():
        s1_ref[...] = jnp.zeros_like(s1_ref)
        sq_ref[...] = jnp.zeros_like(sq_ref)

    row = lax.broadcasted_iota(_i32, (RB, 1), 0) + i * RB
    hdm = jnp.where(row < N, hd, 0.0)
    s1_ref[...] += hdm.sum(axis=0, keepdims=True)
    sq_ref[...] += (hdm * hdm).sum(axis=0, keepdims=True)


def _k5a_call(S2, dp2, Wd, bdr, b2r):
    return pl.pallas_call(
        _k5a_body,
        grid=(NBLK,),
        in_specs=[
            pl.BlockSpec((NC, RB, D2), lambda i: (0, i, 0)),
            pl.BlockSpec((NC, RB, 16), lambda i: (0, i, 0)),
            pl.BlockSpec((D2, D1), lambda i: (0, 0)),
            pl.BlockSpec((1, D1), lambda i: (0, 0)),
            pl.BlockSpec((1, D2), lambda i: (0, 0)),
        ],
        out_specs=[
            pl.BlockSpec((RB, D2), lambda i: (i, 0)),
            pl.BlockSpec((RB, D1), lambda i: (i, 0)),
            pl.BlockSpec((1, D1), lambda i: (0, 0)),
            pl.BlockSpec((1, D1), lambda i: (0, 0)),
        ],
        out_shape=[
            jax.ShapeDtypeStruct((NP, D2), _f32),
            jax.ShapeDtypeStruct((NP, D1), _f32),
            jax.ShapeDtypeStruct((1, D1), _f32),
            jax.ShapeDtypeStruct((1, D1), _f32),
        ],
    )(S2, dp2, Wd, bdr, b2r)


def _k5c_body(hd_ref, s1_ref, sq_ref, g_ref, be_ref,
              wm_ref, bm_ref, wdi_ref, bdi_ref, wp_ref, bp_ref,
              mean_ref, disp_ref, pi_ref):
    mu = s1_ref[...] * (1.0 / N)
    var = sq_ref[...] * (1.0 / N) - mu * mu
    scale = lax.rsqrt(var + 1e-5) * g_ref[...]
    hn = (hd_ref[...] - mu) * scale + be_ref[...]
    hn = jnp.maximum(hn, 0.0)
    m = jnp.dot(hn, wm_ref[...], preferred_element_type=_f32) + bm_ref[...]
    mean_ref[...] = jnp.clip(jnp.exp(m), 1e-5, 1e6)
    d = jnp.dot(hn, wdi_ref[...], preferred_element_type=_f32) + bdi_ref[...]
    sp = jnp.maximum(d, 0.0) + jnp.log(1.0 + jnp.exp(-jnp.abs(d)))
    disp_ref[...] = jnp.clip(sp, 1e-4, 1e4)
    p = jnp.dot(hn, wp_ref[...], preferred_element_type=_f32) + bp_ref[...]
    pi_ref[...] = 1.0 / (1.0 + jnp.exp(-p))


def _k5c_call(hd, s1, sq, gr, ber, Wm, bmr, Wdi, bdir, Wp, bpr):
    return pl.pallas_call(
        _k5c_body,
        grid=(NBLK,),
        in_specs=[
            pl.BlockSpec((RB, D1), lambda i: (i, 0)),
            pl.BlockSpec((1, D1), lambda i: (0, 0)),
            pl.BlockSpec((1, D1), lambda i: (0, 0)),
            pl.BlockSpec((1, D1), lambda i: (0, 0)),
            pl.BlockSpec((1, D1), lambda i: (0, 0)),
            pl.BlockSpec((D1, G), lambda i: (0, 0)),
            pl.BlockSpec((1, G), lambda i: (0, 0)),
            pl.BlockSpec((D1, G), lambda i: (0, 0)),
            pl.BlockSpec((1, G), lambda i: (0, 0)),
            pl.BlockSpec((D1, G), lambda i: (0, 0)),
            pl.BlockSpec((1, G), lambda i: (0, 0)),
        ],
        out_specs=[pl.BlockSpec((RB, G), lambda i: (i, 0))] * 3,
        out_shape=[jax.ShapeDtypeStruct((NP, G), _f32)] * 3,
    )(hd, s1, sq, gr, ber, Wm, bmr, Wdi, bdir, Wp, bpr)


_MESH = plsc.VectorSubcoreMesh(core_axis_name="c", subcore_axis_name="s")
_Z16 = None


def _wid():
    return lax.axis_index("s") * NC + lax.axis_index("c")


def _edge_w_kernel(a1t_hbm, src_hbm, dst_hbm, *rest):
    w_hbms = rest[:H]
    atabs = rest[H:H + 2 * H]
    srcv, dstv = rest[H + 2 * H:H + 2 * H + 2]
    wvs = rest[H + 2 * H + 2:]
    base = _wid() * EPT
    for t in range(2 * H):
        pltpu.sync_copy(a1t_hbm.at[t], atabs[t])

    def batch(j, carry):
        b0 = base + j * EB1
        pltpu.sync_copy(src_hbm.at[pl.ds(b0, EB1)], srcv)
        pltpu.sync_copy(dst_hbm.at[pl.ds(b0, EB1)], dstv)

        def grp(k, c2):
            s16 = srcv[pl.ds(k * 16, 16)]
            d16 = dstv[pl.ds(k * 16, 16)]
            for h in range(H):
                av = plsc.load_gather(atabs[h], [s16])
                bv = plsc.load_gather(atabs[h + H], [d16])
                t = av + bv
                w = jnp.exp(jnp.maximum(t, 0.2 * t))
                wvs[h][pl.ds(k * 16, 16)] = w
            return c2

        lax.fori_loop(0, EB1 // 16, grp, 0)
        for h in range(H):
            pltpu.sync_copy(wvs[h], w_hbms[h].at[pl.ds(b0, EB1)])
        return carry

    lax.fori_loop(0, EPT // EB1, batch, 0)


def _edge_w_call(a1t, src, dst):
    return pl.kernel(
        _edge_w_kernel,
        out_type=[jax.ShapeDtypeStruct((E,), _f32)] * H,
        mesh=_MESH,
        compiler_params=pltpu.CompilerParams(needs_layout_passes=False, use_tc_tiling_on_sc=False),
        scratch_types=[pltpu.VMEM((NP,), _f32)] * (2 * H) + [
            pltpu.VMEM((EB1,), _i32),
            pltpu.VMEM((EB1,), _i32),
        ] + [pltpu.VMEM((EB1,), _f32)] * H,
    )(a1t, src, dst)


def _zero_rows(ref, nrows, width):
    z = jnp.zeros((16,), _f32)

    def body(i, c):
        for k in range(width // 16):
            ref[i, pl.ds(k * 16, 16)] = z
        return c

    lax.fori_loop(0, nrows, body, 0)


def _agg1_kernel(*refs):
    h_refs = refs[:NCHUNK]
    src_hbm, dst_hbm = refs[NCHUNK:NCHUNK + 2]
    w_hbms = refs[NCHUNK + 2:NCHUNK + 2 + H]
    s_refs = refs[NCHUNK + 2 + H:2 * NCHUNK + 2 + H]
    dp_hbm = refs[2 * NCHUNK + 2 + H]
    rest = refs[2 * NCHUNK + 3 + H:]
    srcv_all, dstv_all = rest[0], rest[1]
    rows = rest[2:4]
    dsts = rest[4:6]
    wsm = rest[6:6 + H]
    sems = rest[6 + H:8 + H]
    ssems = rest[8 + H:10 + H]

    cid = lax.axis_index("c")
    sid = lax.axis_index("s")
    ebase = (sid * NC + cid) * EPT
    span0 = sid * ROWS_T
    NB = EPT // EB2

    pltpu.sync_copy(src_hbm.at[pl.ds(ebase, EPT)], srcv_all)
    pltpu.sync_copy(dst_hbm.at[pl.ds(ebase, EPT)], dstv_all)

    def _zero_fill():
        _zero_rows(rows[1], EB2, CHUNK)
        for r in range(ROWS_T // EB2):
            pltpu.sync_copy(rows[1], acc.at[pl.ds(span0 + r * EB2, EB2)])

    acc = rest[10 + H]
    _zero_fill()
    plsc.subcore_barrier()

    def _stage_dst(buf, j):
        for k in range(EB2 // 16):
            dsts[buf][pl.ds(k * 16, 16)] = \
                dstv_all[pl.ds(j * EB2 + k * 16, 16)]

    for c in range(NCHUNK + 1):
        if c < NCHUNK:
            head = c // (NCHUNK // H)
            h_ref = h_refs[c]
            s_ref = s_refs[c]

            def issue(j, buf):
                pltpu.async_copy(
                    w_hbms[head].at[pl.ds(ebase + j * EB2, EB2)],
                    wsm[buf], sems[buf])
                pltpu.async_copy(
                    h_ref.at[srcv_all.at[pl.ds(j * EB2, EB2)]],
                    rows[buf], sems[buf])

            def process(j, buf):
                pltpu.make_async_copy(
                    w_hbms[head].at[pl.ds(ebase + j * EB2, EB2)],
                    wsm[buf], sems[buf]).wait()
                pltpu.make_async_copy(
                    h_ref.at[srcv_all.at[pl.ds(j * EB2, EB2)]],
                    rows[buf], sems[buf]).wait()
                _stage_dst(buf, j)

                def mul_edge(bb, c2):
                    wb = plsc.load_gather(
                        wsm[buf], [jnp.full((16,), bb, _i32)])
                    for f in range(CHUNK // 16):
                        rows[buf][bb, pl.ds(f * 16, 16)] = \
                            rows[buf][bb, pl.ds(f * 16, 16)] * wb
                    return c2

                lax.fori_loop(0, EB2, mul_edge, 0)
                pltpu.sync_copy(rows[buf], acc.at[dsts[buf]], add=True)

            issue(0, 0)

            def pair(k, c2):
                issue(2 * k + 1, 1)
                process(2 * k, 0)
                issue(2 * k + 2, 0)
                process(2 * k + 1, 1)
                return c2

            lax.fori_loop(0, (NB - 1) // 2, pair, 0)
            process(NB - 1, 0)
        else:
            def dbatch(j, c2):
                for h in range(H):
                    pltpu.async_copy(
                        w_hbms[h].at[pl.ds(ebase + j * EB2, EB2)],
                        wsm[h], sems[0])
                for h in range(H):
                    pltpu.make_async_copy(
                        w_hbms[h].at[pl.ds(ebase + j * EB2, EB2)],
                        wsm[h], sems[0]).wait()
                _stage_dst(0, j)

                def wedge(bb, c3):
                    for h in range(H):
                        wb = plsc.load_gather(
                            wsm[h], [jnp.full((16,), bb, _i32)])
                        rows[0][bb, pl.ds(h * 32, 16)] = wb
                        rows[0][bb, pl.ds(h * 32 + 16, 16)] = wb
                    return c3

                lax.fori_loop(0, EB2, wedge, 0)
                pltpu.sync_copy(rows[0], acc.at[dsts[0]], add=True)
                return c2

            lax.fori_loop(0, NB, dbatch, 0)
            s_ref = dp_hbm

        plsc.subcore_barrier()
        pltpu.sync_copy(acc.at[pl.ds(span0, ROWS_T)],
                        s_ref.at[cid, pl.ds(span0, ROWS_T)])
        _zero_fill()
        plsc.subcore_barrier()


def _agg1_call(h_chunks, src, dst, w4):
    return pl.kernel(
        _agg1_kernel,
        out_type=[jax.ShapeDtypeStruct((NC, NP, CHUNK), _f32)] * NCHUNK
        + [jax.ShapeDtypeStruct((NC, NP, CHUNK), _f32)],
        mesh=_MESH,
        compiler_params=pltpu.CompilerParams(needs_layout_passes=False,
                                             use_tc_tiling_on_sc=False),
        scratch_types=[
            pltpu.VMEM((EPT,), _i32),
            pltpu.VMEM((EPT,), _i32),
            pltpu.VMEM((EB2, CHUNK), _f32),
            pltpu.VMEM((EB2, CHUNK), _f32),
            pltpu.VMEM((EB2,), _i32),
            pltpu.VMEM((EB2,), _i32),
        ] + [pltpu.VMEM((EB2,), _f32)] * H + [
            pltpu.SemaphoreType.DMA,
            pltpu.SemaphoreType.DMA,
            pltpu.SemaphoreType.DMA,
            pltpu.SemaphoreType.DMA,
            pltpu.VMEM_SHARED((NP, CHUNK), _f32),
        ],
    )(*h_chunks, src, dst, *w4)


def _agg2_kernel(h2_hbm, a2s_hbm, a2d_hbm, src_hbm, dst_hbm,
                 s2_hbm, dp2_hbm,
                 a2sv, a2dv, srcv_all, dstv_all, rows0, rows1, dst0, dst1,
                 wv, wrows, acc, dden, sem0, sem1):
    rows = (rows0, rows1)
    dsts = (dst0, dst1)
    sems = (sem0, sem1)
    cid = lax.axis_index("c")
    sid = lax.axis_index("s")
    ebase = (sid * NC + cid) * EPT
    span0 = sid * ROWS_T
    NB = EPT // EB2

    pltpu.sync_copy(a2s_hbm, a2sv)
    pltpu.sync_copy(a2d_hbm, a2dv)
    pltpu.sync_copy(src_hbm.at[pl.ds(ebase, EPT)], srcv_all)
    pltpu.sync_copy(dst_hbm.at[pl.ds(ebase, EPT)], dstv_all)

    _zero_rows(rows1, EB2, D2)
    _zero_rows(wrows, EB2, 16)
    for r in range(ROWS_T // EB2):
        pltpu.sync_copy(rows1, acc.at[pl.ds(span0 + r * EB2, EB2)])
        pltpu.sync_copy(wrows, dden.at[pl.ds(span0 + r * EB2, EB2)])
    plsc.subcore_barrier()

    def issue(j, buf):
        pltpu.async_copy(h2_hbm.at[srcv_all.at[pl.ds(j * EB2, EB2)]],
                         rows[buf], sems[buf])

    def process(j, buf):
        def grp(k, c2):
            s16 = srcv_all[pl.ds(j * EB2 + k * 16, 16)]
            d16 = dstv_all[pl.ds(j * EB2 + k * 16, 16)]
            av = plsc.load_gather(a2sv, [s16])
            bv = plsc.load_gather(a2dv, [d16])
            t = av + bv
            wv[pl.ds(k * 16, 16)] = jnp.exp(jnp.maximum(t, 0.2 * t))
            dsts[buf][pl.ds(k * 16, 16)] = d16
            return c2

        lax.fori_loop(0, EB2 // 16, grp, 0)
        pltpu.make_async_copy(h2_hbm.at[srcv_all.at[pl.ds(j * EB2, EB2)]],
                              rows[buf], sems[buf]).wait()

        def mul_edge(bb, c2):
            wb = plsc.load_gather(wv, [jnp.full((16,), bb, _i32)])
            for f in range(D2 // 16):
                rows[buf][bb, pl.ds(f * 16, 16)] = \
                    rows[buf][bb, pl.ds(f * 16, 16)] * wb
            wrows[bb] = wb
            return c2

        lax.fori_loop(0, EB2, mul_edge, 0)
        pltpu.sync_copy(rows[buf], acc.at[dsts[buf]], add=True)
        pltpu.sync_copy(wrows, dden.at[dsts[buf]], add=True)

    issue(0, 0)

    def pair(k, c2):
        issue(2 * k + 1, 1)
        process(2 * k, 0)
        issue(2 * k + 2, 0)
        process(2 * k + 1, 1)
        return c2

    lax.fori_loop(0, (NB - 1) // 2, pair, 0)
    process(NB - 1, 0)

    plsc.subcore_barrier()
    pltpu.sync_copy(acc.at[pl.ds(span0, ROWS_T)],
                    s2_hbm.at[cid, pl.ds(span0, ROWS_T)])
    pltpu.sync_copy(dden.at[pl.ds(span0, ROWS_T)],
                    dp2_hbm.at[cid, pl.ds(span0, ROWS_T)])


def _agg2_call(h2pre, a2s, a2d, src, dst):
    return pl.kernel(
        _agg2_kernel,
        out_type=[
            jax.ShapeDtypeStruct((NC, NP, D2), _f32),
            jax.ShapeDtypeStruct((NC, NP, 16), _f32),
        ],
        mesh=_MESH,
        compiler_params=pltpu.CompilerParams(needs_layout_passes=False,
                                             use_tc_tiling_on_sc=False),
        scratch_types=[
            pltpu.VMEM((NP,), _f32),
            pltpu.VMEM((NP,), _f32),
            pltpu.VMEM((EPT,), _i32),
            pltpu.VMEM((EPT,), _i32),
            pltpu.VMEM((EB2, D2), _f32),
            pltpu.VMEM((EB2, D2), _f32),
            pltpu.VMEM((EB2,), _i32),
            pltpu.VMEM((EB2,), _i32),
            pltpu.VMEM((EB2,), _f32),
            pltpu.VMEM((EB2, 16), _f32),
            pltpu.VMEM_SHARED((NP, D2), _f32),
            pltpu.VMEM_SHARED((NP, 16), _f32),
            pltpu.SemaphoreType.DMA,
            pltpu.SemaphoreType.DMA,
        ],
    )(h2pre, a2s, a2d, src, dst)


def kernel(x, edge_index, W1, att_src1, att_dst1, b1, W2, att_src2, att_dst2,
           b2, Wd, bd, gamma, beta, Wm, bm, Wdi, bdi, Wp, bp):
    src = edge_index[0]
    dst = edge_index[1]

    eyeH = jnp.eye(H, dtype=_f32)
    a_s = (eyeH[:, None, :] * att_src1[:, :, None]).reshape(F1, H)
    a_d = (eyeH[:, None, :] * att_dst1[:, :, None]).reshape(F1, H)
    A1 = jnp.concatenate([a_s, a_d, jnp.zeros((F1, 8), _f32)], axis=1)
    At2 = jnp.concatenate(
        [att_src2.T, att_dst2.T, jnp.zeros((D2, 14), _f32)], axis=1)

    x_pad = jnp.pad(x, ((0, NP - N), (0, 0)))
    *h_chunks, a1t = _k1_call(x_pad, W1, A1)
    w4 = _edge_w_call(a1t, src, dst)
    *s_chunks, dp = _agg1_call(h_chunks, src, dst, w4)
    h2pre = _k3_call(s_chunks, dp, W2, b1.reshape(1, F1))
    a2t = _att_proj_call(h2pre, At2)
    S2, dp2 = _agg2_call(h2pre, a2t[0], a2t[1], src, dst)
    rep, hd, s1, sq = _k5a_call(S2, dp2, Wd, bd.reshape(1, D1),
                                b2.reshape(1, D2))
    mean, disp, pi = _k5c_call(
        hd, s1, sq, gamma.reshape(1, D1), beta.reshape(1, D1),
        Wm, bm.reshape(1, G), Wdi, bdi.reshape(1, G), Wp, bp.reshape(1, G))
    return (mean[:N], disp[:N], pi[:N], rep[:N])


# speedup vs baseline: 1.1845x; 1.0304x over previous
import functools

import jax
import jax.numpy as jnp
from jax import lax
from jax.experimental import pallas as pl
from jax.experimental.pallas import tpu as pltpu
from jax.experimental.pallas import tpu_sc as plsc

N = 10000
NP = 10240
E = 320000
G = 128
H = 4
D1 = 256
D2 = 64
F1 = H * D1
CHUNK = 128
NCHUNK = F1 // CHUNK

NC = 2
NS = 16
NW = NC * NS
EPT = E // NW
ROWS_T = NP // NS
ZR = 128

RB = 1024
NBLK = NP // RB

EB1 = 2000
EB2 = 80

_f32 = jnp.float32
_i32 = jnp.int32


def _k1_body(x_ref, w1_ref, a1_ref, *out_refs):
    xb = x_ref[...]
    h = jnp.dot(xb, w1_ref[...], preferred_element_type=_f32)
    for c in range(NCHUNK):
        out_refs[c][...] = h[:, c * CHUNK:(c + 1) * CHUNK]
    out_refs[NCHUNK][...] = lax.dot_general(
        a1_ref[...], h, (((0,), (1,)), ((), ())),
        preferred_element_type=_f32, precision=lax.Precision.HIGHEST)


def _k1_call(x, W1, A1):
    return pl.pallas_call(
        _k1_body,
        grid=(NBLK,),
        in_specs=[
            pl.BlockSpec((RB, G), lambda i: (i, 0)),
            pl.BlockSpec((G, F1), lambda i: (0, 0)),
            pl.BlockSpec((F1, 16), lambda i: (0, 0)),
        ],
        out_specs=[pl.BlockSpec((RB, CHUNK), lambda i: (i, 0))] * NCHUNK
        + [pl.BlockSpec((16, RB), lambda i: (0, i))],
        out_shape=[jax.ShapeDtypeStruct((NP, CHUNK), _f32)] * NCHUNK
        + [jax.ShapeDtypeStruct((16, NP), _f32)],
    )(x, W1, A1)


def _att_proj_body(x_ref, va_ref, out_ref):
    out_ref[...] = lax.dot_general(
        va_ref[...], x_ref[...], (((0,), (1,)), ((), ())),
        preferred_element_type=_f32, precision=lax.Precision.HIGHEST)


def _att_proj_call(x, Va):
    k = x.shape[1]
    return pl.pallas_call(
        _att_proj_body,
        grid=(1,),
        in_specs=[
            pl.BlockSpec((NP, k), lambda i: (0, 0)),
            pl.BlockSpec((k, 16), lambda i: (0, 0)),
        ],
        out_specs=pl.BlockSpec((16, NP), lambda i: (0, 0)),
        out_shape=jax.ShapeDtypeStruct((16, NP), _f32),
    )(x, Va)


def _k3_body(*refs):
    s_refs = refs[:NCHUNK]
    dp_ref, w2_ref, b1_ref, h2pre_ref = refs[NCHUNK:]
    dpv = dp_ref[0] + dp_ref[1]
    parts = []
    for c in range(NCHUNK):
        sc = s_refs[c][0] + s_refs[c][1]
        h = c // (NCHUNK // H)
        deninv = 1.0 / (dpv[:, h * 32] + 1e-16)
        parts.append(sc * deninv[:, None])
    h1 = jnp.concatenate(parts, axis=1) + b1_ref[...]
    h1 = jnp.where(h1 > 0, h1, jnp.exp(jnp.minimum(h1, 0.0)) - 1.0)
    h2pre_ref[...] = jnp.dot(h1, w2_ref[...], preferred_element_type=_f32)


def _k3_call(s_chunks, dp, W2, b1r):
    return pl.pallas_call(
        _k3_body,
        grid=(NBLK,),
        in_specs=[pl.BlockSpec((NC, RB, CHUNK), lambda i: (0, i, 0))] * NCHUNK
        + [
            pl.BlockSpec((NC, RB, CHUNK), lambda i: (0, i, 0)),
            pl.BlockSpec((F1, D2), lambda i: (0, 0)),
            pl.BlockSpec((1, F1), lambda i: (0, 0)),
        ],
        out_specs=pl.BlockSpec((RB, D2), lambda i: (i, 0)),
        out_shape=jax.ShapeDtypeStruct((NP, D2), _f32),
    )(*s_chunks, dp, W2, b1r)


def _k5a_body(s2_ref, dp2_ref, wd_ref, bd_ref, b2_ref,
              rep_ref, hd_ref, s1_ref, sq_ref):
    den = dp2_ref[...].sum(axis=(0, 2)) * (1.0 / 16.0)
    ssum = s2_ref[0] + s2_ref[1]
    rep = ssum * (1.0 / (den + 1e-16))[:, None] + b2_ref[...]
    rep_ref[...] = rep
    hd = jnp.dot(rep, wd_ref[...], preferred_element_type=_f32) + bd_ref[...]
    hd_ref[...] = hd
    i = pl.program_id(0)

    @pl.when(i == 0)
    def _():
        s1_ref[...] = jnp.zeros_like(s1_ref)
        sq_ref[...] = jnp.zeros_like(sq_ref)

    row = lax.broadcasted_iota(_i32, (RB, 1), 0) + i * RB
    hdm = jnp.where(row < N, hd, 0.0)
    s1_ref[...] += hdm.sum(axis=0, keepdims=True)
    sq_ref[...] += (hdm * hdm).sum(axis=0, keepdims=True)


def _k5a_call(S2, dp2, Wd, bdr, b2r):
    return pl.pallas_call(
        _k5a_body,
        grid=(NBLK,),
        in_specs=[
            pl.BlockSpec((NC, RB, D2), lambda i: (0, i, 0)),
            pl.BlockSpec((NC, RB, 16), lambda i: (0, i, 0)),
            pl.BlockSpec((D2, D1), lambda i: (0, 0)),
            pl.BlockSpec((1, D1), lambda i: (0, 0)),
            pl.BlockSpec((1, D2), lambda i: (0, 0)),
        ],
        out_specs=[
            pl.BlockSpec((RB, D2), lambda i: (i, 0)),
            pl.BlockSpec((RB, D1), lambda i: (i, 0)),
            pl.BlockSpec((1, D1), lambda i: (0, 0)),
            pl.BlockSpec((1, D1), lambda i: (0, 0)),
        ],
        out_shape=[
            jax.ShapeDtypeStruct((NP, D2), _f32),
            jax.ShapeDtypeStruct((NP, D1), _f32),
            jax.ShapeDtypeStruct((1, D1), _f32),
            jax.ShapeDtypeStruct((1, D1), _f32),
        ],
    )(S2, dp2, Wd, bdr, b2r)


def _k5c_body(hd_ref, s1_ref, sq_ref, g_ref, be_ref,
              wm_ref, bm_ref, wdi_ref, bdi_ref, wp_ref, bp_ref,
              mean_ref, disp_ref, pi_ref):
    mu = s1_ref[...] * (1.0 / N)
    var = sq_ref[...] * (1.0 / N) - mu * mu
    scale = lax.rsqrt(var + 1e-5) * g_ref[...]
    hn = (hd_ref[...] - mu) * scale + be_ref[...]
    hn = jnp.maximum(hn, 0.0)
    m = jnp.dot(hn, wm_ref[...], preferred_element_type=_f32) + bm_ref[...]
    mean_ref[...] = jnp.clip(jnp.exp(m), 1e-5, 1e6)
    d = jnp.dot(hn, wdi_ref[...], preferred_element_type=_f32) + bdi_ref[...]
    sp = jnp.maximum(d, 0.0) + jnp.log(1.0 + jnp.exp(-jnp.abs(d)))
    disp_ref[...] = jnp.clip(sp, 1e-4, 1e4)
    p = jnp.dot(hn, wp_ref[...], preferred_element_type=_f32) + bp_ref[...]
    pi_ref[...] = 1.0 / (1.0 + jnp.exp(-p))


def _k5c_call(hd, s1, sq, gr, ber, Wm, bmr, Wdi, bdir, Wp, bpr):
    return pl.pallas_call(
        _k5c_body,
        grid=(NBLK,),
        in_specs=[
            pl.BlockSpec((RB, D1), lambda i: (i, 0)),
            pl.BlockSpec((1, D1), lambda i: (0, 0)),
            pl.BlockSpec((1, D1), lambda i: (0, 0)),
            pl.BlockSpec((1, D1), lambda i: (0, 0)),
            pl.BlockSpec((1, D1), lambda i: (0, 0)),
            pl.BlockSpec((D1, G), lambda i: (0, 0)),
            pl.BlockSpec((1, G), lambda i: (0, 0)),
            pl.BlockSpec((D1, G), lambda i: (0, 0)),
            pl.BlockSpec((1, G), lambda i: (0, 0)),
            pl.BlockSpec((D1, G), lambda i: (0, 0)),
            pl.BlockSpec((1, G), lambda i: (0, 0)),
        ],
        out_specs=[pl.BlockSpec((RB, G), lambda i: (i, 0))] * 3,
        out_shape=[jax.ShapeDtypeStruct((NP, G), _f32)] * 3,
    )(hd, s1, sq, gr, ber, Wm, bmr, Wdi, bdir, Wp, bpr)


_MESH = plsc.VectorSubcoreMesh(core_axis_name="c", subcore_axis_name="s")
_Z16 = None


def _wid():
    return lax.axis_index("s") * NC + lax.axis_index("c")


def _edge_w_kernel(a1t_hbm, src_hbm, dst_hbm, *rest):
    w_hbms = rest[:H]
    atabs = rest[H:H + 2 * H]
    srcv, dstv = rest[H + 2 * H:H + 2 * H + 2]
    wvs = rest[H + 2 * H + 2:]
    base = _wid() * EPT
    for t in range(2 * H):
        pltpu.sync_copy(a1t_hbm.at[t], atabs[t])

    def batch(j, carry):
        b0 = base + j * EB1
        pltpu.sync_copy(src_hbm.at[pl.ds(b0, EB1)], srcv)
        pltpu.sync_copy(dst_hbm.at[pl.ds(b0, EB1)], dstv)

        def grp(k, c2):
            s16 = srcv[pl.ds(k * 16, 16)]
            d16 = dstv[pl.ds(k * 16, 16)]
            for h in range(H):
                av = plsc.load_gather(atabs[h], [s16])
                bv = plsc.load_gather(atabs[h + H], [d16])
                t = av + bv
                w = jnp.exp(jnp.maximum(t, 0.2 * t))
                wvs[h][pl.ds(k * 16, 16)] = w
            return c2

        lax.fori_loop(0, EB1 // 16, grp, 0)
        for h in range(H):
            pltpu.sync_copy(wvs[h], w_hbms[h].at[pl.ds(b0, EB1)])
        return carry

    lax.fori_loop(0, EPT // EB1, batch, 0)


def _edge_w_call(a1t, src, dst):
    return pl.kernel(
        _edge_w_kernel,
        out_type=[jax.ShapeDtypeStruct((E,), _f32)] * H,
        mesh=_MESH,
        compiler_params=pltpu.CompilerParams(needs_layout_passes=False, use_tc_tiling_on_sc=False),
        scratch_types=[pltpu.VMEM((NP,), _f32)] * (2 * H) + [
            pltpu.VMEM((EB1,), _i32),
            pltpu.VMEM((EB1,), _i32),
        ] + [pltpu.VMEM((EB1,), _f32)] * H,
    )(a1t, src, dst)


def _zero_rows(ref, nrows, width):
    z = jnp.zeros((16,), _f32)

    def body(i, c):
        for k in range(width // 16):
            ref[i, pl.ds(k * 16, 16)] = z
        return c

    lax.fori_loop(0, nrows, body, 0)


def _agg1_kernel(*refs):
    h_refs = refs[:NCHUNK]
    src_hbm, dst_hbm = refs[NCHUNK:NCHUNK + 2]
    w_hbms = refs[NCHUNK + 2:NCHUNK + 2 + H]
    s_refs = refs[NCHUNK + 2 + H:2 * NCHUNK + 2 + H]
    dp_hbm = refs[2 * NCHUNK + 2 + H]
    rest = refs[2 * NCHUNK + 3 + H:]
    srcv_all, dstv_all = rest[0], rest[1]
    rows = rest[2:4]
    dsts = rest[4:6]
    wsm = rest[6:6 + H]
    sems = rest[6 + H:8 + H]
    ssems = rest[8 + H:10 + H]

    cid = lax.axis_index("c")
    sid = lax.axis_index("s")
    ebase = (sid * NC + cid) * EPT
    span0 = sid * ROWS_T
    NB = EPT // EB2

    pltpu.sync_copy(src_hbm.at[pl.ds(ebase, EPT)], srcv_all)
    pltpu.sync_copy(dst_hbm.at[pl.ds(ebase, EPT)], dstv_all)

    def _zero_fill():
        _zero_rows(rows[1], EB2, CHUNK)
        for r in range(ROWS_T // EB2):
            pltpu.sync_copy(rows[1], acc.at[pl.ds(span0 + r * EB2, EB2)])

    acc = rest[10 + H]
    _zero_fill()
    plsc.subcore_barrier()

    def _stage_dst(buf, j):
        for k in range(EB2 // 16):
            dsts[buf][pl.ds(k * 16, 16)] = \
                dstv_all[pl.ds(j * EB2 + k * 16, 16)]

    for c in range(NCHUNK + 1):
        if c < NCHUNK:
            head = c // (NCHUNK // H)
            h_ref = h_refs[c]
            s_ref = s_refs[c]

            def issue(j, buf):
                pltpu.async_copy(
                    w_hbms[head].at[pl.ds(ebase + j * EB2, EB2)],
                    wsm[buf], sems[buf])
                pltpu.async_copy(
                    h_ref.at[srcv_all.at[pl.ds(j * EB2, EB2)]],
                    rows[buf], sems[buf])

            def process(j, buf):
                pltpu.make_async_copy(
                    w_hbms[head].at[pl.ds(ebase + j * EB2, EB2)],
                    wsm[buf], sems[buf]).wait()
                pltpu.make_async_copy(
                    h_ref.at[srcv_all.at[pl.ds(j * EB2, EB2)]],
                    rows[buf], sems[buf]).wait()
                _stage_dst(buf, j)

                def mul_edge(b4, c2):
                    for u in range(4):
                        bb = b4 * 4 + u
                        wb = plsc.load_gather(
                            wsm[buf], [jnp.full((16,), bb, _i32)])
                        for f in range(CHUNK // 16):
                            rows[buf][bb, pl.ds(f * 16, 16)] = \
                                rows[buf][bb, pl.ds(f * 16, 16)] * wb
                    return c2

                lax.fori_loop(0, EB2 // 4, mul_edge, 0)
                pltpu.sync_copy(rows[buf], acc.at[dsts[buf]], add=True)

            issue(0, 0)

            def pair(k, c2):
                issue(2 * k + 1, 1)
                process(2 * k, 0)
                issue(2 * k + 2, 0)
                process(2 * k + 1, 1)
                return c2

            lax.fori_loop(0, (NB - 1) // 2, pair, 0)
            process(NB - 1, 0)
        else:
            def dbatch(j, c2):
                for h in range(H):
                    pltpu.async_copy(
                        w_hbms[h].at[pl.ds(ebase + j * EB2, EB2)],
                        wsm[h], sems[0])
                for h in range(H):
                    pltpu.make_async_copy(
                        w_hbms[h].at[pl.ds(ebase + j * EB2, EB2)],
                        wsm[h], sems[0]).wait()
                _stage_dst(0, j)

                def wedge(b4, c3):
                    for u in range(4):
                        bb = b4 * 4 + u
                        for h in range(H):
                            wb = plsc.load_gather(
                                wsm[h], [jnp.full((16,), bb, _i32)])
                            rows[0][bb, pl.ds(h * 32, 16)] = wb
                            rows[0][bb, pl.ds(h * 32 + 16, 16)] = wb
                    return c3

                lax.fori_loop(0, EB2 // 4, wedge, 0)
                pltpu.sync_copy(rows[0], acc.at[dsts[0]], add=True)
                return c2

            lax.fori_loop(0, NB, dbatch, 0)
            s_ref = dp_hbm

        plsc.subcore_barrier()
        pltpu.sync_copy(acc.at[pl.ds(span0, ROWS_T)],
                        s_ref.at[cid, pl.ds(span0, ROWS_T)])
        _zero_fill()
        plsc.subcore_barrier()


def _agg1_call(h_chunks, src, dst, w4):
    return pl.kernel(
        _agg1_kernel,
        out_type=[jax.ShapeDtypeStruct((NC, NP, CHUNK), _f32)] * NCHUNK
        + [jax.ShapeDtypeStruct((NC, NP, CHUNK), _f32)],
        mesh=_MESH,
        compiler_params=pltpu.CompilerParams(needs_layout_passes=False,
                                             use_tc_tiling_on_sc=False),
        scratch_types=[
            pltpu.VMEM((EPT,), _i32),
            pltpu.VMEM((EPT,), _i32),
            pltpu.VMEM((EB2, CHUNK), _f32),
            pltpu.VMEM((EB2, CHUNK), _f32),
            pltpu.VMEM((EB2,), _i32),
            pltpu.VMEM((EB2,), _i32),
        ] + [pltpu.VMEM((EB2,), _f32)] * H + [
            pltpu.SemaphoreType.DMA,
            pltpu.SemaphoreType.DMA,
            pltpu.SemaphoreType.DMA,
            pltpu.SemaphoreType.DMA,
            pltpu.VMEM_SHARED((NP, CHUNK), _f32),
        ],
    )(*h_chunks, src, dst, *w4)


def _agg2_kernel(h2_hbm, a2s_hbm, a2d_hbm, src_hbm, dst_hbm,
                 s2_hbm, dp2_hbm,
                 a2sv, a2dv, srcv_all, dstv_all, rows0, rows1, dst0, dst1,
                 wv, wrows, acc, dden, sem0, sem1):
    rows = (rows0, rows1)
    dsts = (dst0, dst1)
    sems = (sem0, sem1)
    cid = lax.axis_index("c")
    sid = lax.axis_index("s")
    ebase = (sid * NC + cid) * EPT
    span0 = sid * ROWS_T
    NB = EPT // EB2

    pltpu.sync_copy(a2s_hbm, a2sv)
    pltpu.sync_copy(a2d_hbm, a2dv)
    pltpu.sync_copy(src_hbm.at[pl.ds(ebase, EPT)], srcv_all)
    pltpu.sync_copy(dst_hbm.at[pl.ds(ebase, EPT)], dstv_all)

    _zero_rows(rows1, EB2, D2)
    _zero_rows(wrows, EB2, 16)
    for r in range(ROWS_T // EB2):
        pltpu.sync_copy(rows1, acc.at[pl.ds(span0 + r * EB2, EB2)])
        pltpu.sync_copy(wrows, dden.at[pl.ds(span0 + r * EB2, EB2)])
    plsc.subcore_barrier()

    def issue(j, buf):
        pltpu.async_copy(h2_hbm.at[srcv_all.at[pl.ds(j * EB2, EB2)]],
                         rows[buf], sems[buf])

    def process(j, buf):
        def grp(k, c2):
            s16 = srcv_all[pl.ds(j * EB2 + k * 16, 16)]
            d16 = dstv_all[pl.ds(j * EB2 + k * 16, 16)]
            av = plsc.load_gather(a2sv, [s16])
            bv = plsc.load_gather(a2dv, [d16])
            t = av + bv
            wv[pl.ds(k * 16, 16)] = jnp.exp(jnp.maximum(t, 0.2 * t))
            dsts[buf][pl.ds(k * 16, 16)] = d16
            return c2

        lax.fori_loop(0, EB2 // 16, grp, 0)
        pltpu.make_async_copy(h2_hbm.at[srcv_all.at[pl.ds(j * EB2, EB2)]],
                              rows[buf], sems[buf]).wait()

        def mul_edge(b4, c2):
            for u in range(4):
                bb = b4 * 4 + u
                wb = plsc.load_gather(wv, [jnp.full((16,), bb, _i32)])
                for f in range(D2 // 16):
                    rows[buf][bb, pl.ds(f * 16, 16)] = \
                        rows[buf][bb, pl.ds(f * 16, 16)] * wb
                wrows[bb] = wb
            return c2

        lax.fori_loop(0, EB2 // 4, mul_edge, 0)
        pltpu.sync_copy(rows[buf], acc.at[dsts[buf]], add=True)
        pltpu.sync_copy(wrows, dden.at[dsts[buf]], add=True)

    issue(0, 0)

    def pair(k, c2):
        issue(2 * k + 1, 1)
        process(2 * k, 0)
        issue(2 * k + 2, 0)
        process(2 * k + 1, 1)
        return c2

    lax.fori_loop(0, (NB - 1) // 2, pair, 0)
    process(NB - 1, 0)

    plsc.subcore_barrier()
    pltpu.sync_copy(acc.at[pl.ds(span0, ROWS_T)],
                    s2_hbm.at[cid, pl.ds(span0, ROWS_T)])
    pltpu.sync_copy(dden.at[pl.ds(span0, ROWS_T)],
                    dp2_hbm.at[cid, pl.ds(span0, ROWS_T)])


def _agg2_call(h2pre, a2s, a2d, src, dst):
    return pl.kernel(
        _agg2_kernel,
        out_type=[
            jax.ShapeDtypeStruct((NC, NP, D2), _f32),
            jax.ShapeDtypeStruct((NC, NP, 16), _f32),
        ],
        mesh=_MESH,
        compiler_params=pltpu.CompilerParams(needs_layout_passes=False,
                                             use_tc_tiling_on_sc=False),
        scratch_types=[
            pltpu.VMEM((NP,), _f32),
            pltpu.VMEM((NP,), _f32),
            pltpu.VMEM((EPT,), _i32),
            pltpu.VMEM((EPT,), _i32),
            pltpu.VMEM((EB2, D2), _f32),
            pltpu.VMEM((EB2, D2), _f32),
            pltpu.VMEM((EB2,), _i32),
            pltpu.VMEM((EB2,), _i32),
            pltpu.VMEM((EB2,), _f32),
            pltpu.VMEM((EB2, 16), _f32),
            pltpu.VMEM_SHARED((NP, D2), _f32),
            pltpu.VMEM_SHARED((NP, 16), _f32),
            pltpu.SemaphoreType.DMA,
            pltpu.SemaphoreType.DMA,
        ],
    )(h2pre, a2s, a2d, src, dst)


def kernel(x, edge_index, W1, att_src1, att_dst1, b1, W2, att_src2, att_dst2,
           b2, Wd, bd, gamma, beta, Wm, bm, Wdi, bdi, Wp, bp):
    src = edge_index[0]
    dst = edge_index[1]

    eyeH = jnp.eye(H, dtype=_f32)
    a_s = (eyeH[:, None, :] * att_src1[:, :, None]).reshape(F1, H)
    a_d = (eyeH[:, None, :] * att_dst1[:, :, None]).reshape(F1, H)
    A1 = jnp.concatenate([a_s, a_d, jnp.zeros((F1, 8), _f32)], axis=1)
    At2 = jnp.concatenate(
        [att_src2.T, att_dst2.T, jnp.zeros((D2, 14), _f32)], axis=1)

    x_pad = jnp.pad(x, ((0, NP - N), (0, 0)))
    *h_chunks, a1t = _k1_call(x_pad, W1, A1)
    w4 = _edge_w_call(a1t, src, dst)
    *s_chunks, dp = _agg1_call(h_chunks, src, dst, w4)
    h2pre = _k3_call(s_chunks, dp, W2, b1.reshape(1, F1))
    a2t = _att_proj_call(h2pre, At2)
    S2, dp2 = _agg2_call(h2pre, a2t[0], a2t[1], src, dst)
    rep, hd, s1, sq = _k5a_call(S2, dp2, Wd, bd.reshape(1, D1),
                                b2.reshape(1, D2))
    mean, disp, pi = _k5c_call(
        hd, s1, sq, gamma.reshape(1, D1), beta.reshape(1, D1),
        Wm, bm.reshape(1, G), Wdi, bdi.reshape(1, G), Wp, bp.reshape(1, G))
    return (mean[:N], disp[:N], pi[:N], rep[:N])
